# baseline (speedup 1.0000x reference)
import functools

import jax
import jax.numpy as jnp
from jax import lax
from jax.experimental import pallas as pl
from jax.experimental.pallas import tpu as pltpu
from jax.experimental.pallas import tpu_sc as plsc

N = 10000
K = 32
D = 128

N_SC = 2048
N_TC = N - N_SC
BN = 512
BN2 = 512

NC = 2
NS = 16
NW = NC * NS
ROWS_W = N_SC // NW
RCH = 8
NCHUNK = ROWS_W // RCH


def _fused_kernel(x_ref, n_ref, wlt_ref, wrt_ref, b_ref, o_ref):
    nsum = jnp.sum(n_ref[...], axis=1)
    acc = jnp.dot(x_ref[...], wlt_ref[...], preferred_element_type=jnp.float32)
    acc += jnp.dot(nsum * (1.0 / K), wrt_ref[...], preferred_element_type=jnp.float32)
    o_ref[...] = acc + b_ref[...]


def _tail_kernel(x_ref, ns_ref, wlt_ref, wrt_ref, b_ref, alias_ref, o_ref):
    del alias_ref
    acc = jnp.dot(x_ref[...], wlt_ref[...], preferred_element_type=jnp.float32)
    acc += jnp.dot(ns_ref[...] * (1.0 / K), wrt_ref[...], preferred_element_type=jnp.float32)
    o_ref[...] = acc + b_ref[...]


def _sc_body(neigh_hbm, out_hbm, buf, obuf):
    c = lax.axis_index("c")
    s = lax.axis_index("s")
    wid = s * NC + c
    row0 = wid * ROWS_W

    def chunk_body(ch, carry):
        base = pl.multiple_of(row0 + ch * RCH, 8)
        pltpu.sync_copy(neigh_hbm.at[pl.ds(base, RCH)], buf)

        def row_body(r, carry2):
            for c16 in range(D // 16):
                sl = pl.ds(c16 * 16, 16)
                acc = buf[r, 0, sl]
                for k in range(1, K):
                    acc = acc + buf[r, k, sl]
                obuf[r, sl] = acc
            return carry2

        lax.fori_loop(0, RCH, row_body, 0)
        pltpu.sync_copy(obuf, out_hbm.at[pl.ds(base, RCH)])
        return carry

    lax.fori_loop(0, NCHUNK, chunk_body, 0)


_sc_mean = functools.partial(
    pl.kernel,
    mesh=plsc.VectorSubcoreMesh(core_axis_name="c", subcore_axis_name="s"),
    out_type=jax.ShapeDtypeStruct((N_SC, D), jnp.float32),
    scratch_types=[
        pltpu.VMEM((RCH, K, D), jnp.float32),
        pltpu.VMEM((RCH, D), jnp.float32),
    ],
)(_sc_body)


@jax.jit
def kernel(x, neigh_x, W_l, b_l, W_r, b_r):
    wlt = W_l.T
    wrt = W_r.T
    b = (b_l + b_r).reshape(1, D)

    nsum_sc = _sc_mean(neigh_x)

    out1 = pl.pallas_call(
        _fused_kernel,
        grid=(pl.cdiv(N_TC, BN),),
        in_specs=[
            pl.BlockSpec((BN, D), lambda i: (N_SC // BN + i, 0)),
            pl.BlockSpec((BN, K, D), lambda i: (N_SC // BN + i, 0, 0)),
            pl.BlockSpec((D, D), lambda i: (0, 0)),
            pl.BlockSpec((D, D), lambda i: (0, 0)),
            pl.BlockSpec((1, D), lambda i: (0, 0)),
        ],
        out_specs=pl.BlockSpec((BN, D), lambda i: (N_SC // BN + i, 0)),
        out_shape=jax.ShapeDtypeStruct((N, D), jnp.float32),
    )(x, neigh_x, wlt, wrt, b)

    out = pl.pallas_call(
        _tail_kernel,
        grid=(N_SC // BN2,),
        in_specs=[
            pl.BlockSpec((BN2, D), lambda j: (j, 0)),
            pl.BlockSpec((BN2, D), lambda j: (j, 0)),
            pl.BlockSpec((D, D), lambda j: (0, 0)),
            pl.BlockSpec((D, D), lambda j: (0, 0)),
            pl.BlockSpec((1, D), lambda j: (0, 0)),
            pl.BlockSpec(memory_space=pl.ANY),
        ],
        out_specs=pl.BlockSpec((BN2, D), lambda j: (j, 0)),
        out_shape=jax.ShapeDtypeStruct((N, D), jnp.float32),
        input_output_aliases={5: 0},
    )(x, nsum_sc, wlt, wrt, b, out1)
    return out


# speedup vs baseline: 1.3249x; 1.3249x over previous
import jax
import jax.numpy as jnp
from jax.experimental import pallas as pl

N = 10000
K = 32
D = 128
BN = 400


def _fused_kernel(x_ref, n_ref, wlt_ref, wrt_ref, b_ref, o_ref):
    nsum = jnp.sum(n_ref[...], axis=1)
    acc = jnp.dot(x_ref[...], wlt_ref[...], preferred_element_type=jnp.float32)
    acc += jnp.dot(nsum * (1.0 / K), wrt_ref[...], preferred_element_type=jnp.float32)
    o_ref[...] = acc + b_ref[...]


@jax.jit
def kernel(x, neigh_x, W_l, b_l, W_r, b_r):
    wlt = W_l.T
    wrt = W_r.T
    b = (b_l + b_r).reshape(1, D)
    grid = (pl.cdiv(N, BN),)
    return pl.pallas_call(
        _fused_kernel,
        grid=grid,
        in_specs=[
            pl.BlockSpec((BN, D), lambda i: (i, 0)),
            pl.BlockSpec((BN, K, D), lambda i: (i, 0, 0)),
            pl.BlockSpec((D, D), lambda i: (0, 0)),
            pl.BlockSpec((D, D), lambda i: (0, 0)),
            pl.BlockSpec((1, D), lambda i: (0, 0)),
        ],
        out_specs=pl.BlockSpec((BN, D), lambda i: (i, 0)),
        out_shape=jax.ShapeDtypeStruct((N, D), jnp.float32),
    )(x, neigh_x, wlt, wrt, b)
